# baseline (speedup 1.0000x reference)
import jax
import jax.numpy as jnp
from jax import lax
from jax.experimental import pallas as pl
from jax.experimental.pallas import tpu as pltpu

_N_ROWS = 1024
_N_COLS = 100000
_TOPK = 5

_RB = 64
_NRB = _N_ROWS // _RB


def _count_body(x_ref, tgt_ref, out_ref):
    r = pl.program_id(0)

    @pl.when(r == 0)
    def _():
        out_ref[...] = jnp.zeros_like(out_ref)

    tgt = tgt_ref[...]
    x = x_ref[...]
    cols = lax.broadcasted_iota(jnp.int32, (_RB, _N_COLS), 1)
    t = jnp.sum(jnp.where(cols == tgt, x, 0.0), axis=1, keepdims=True)
    before = jnp.where(cols < tgt, 1.0, 0.0)
    ahead = jnp.where(x > t, 1.0, jnp.where(x == t, before, 0.0))
    rank = jnp.sum(ahead, axis=1, keepdims=True)
    correct = jnp.where(rank < float(_TOPK), 1.0, 0.0)
    out_ref[...] += jnp.sum(correct).reshape(1, 1) * (100.0 / _N_ROWS)


_count = pl.pallas_call(
    _count_body,
    grid=(_NRB,),
    in_specs=[
        pl.BlockSpec((_RB, _N_COLS), lambda r: (r, 0)),
        pl.BlockSpec((_RB, 1), lambda r: (r, 0)),
    ],
    out_specs=pl.BlockSpec((1, 1), lambda r: (0, 0)),
    out_shape=jax.ShapeDtypeStruct((1, 1), jnp.float32),
)


def kernel(output, target):
    tgt = target.astype(jnp.int32)
    res = _count(output, tgt.reshape(_N_ROWS, 1))
    return res.reshape(1)
